# baseline (speedup 1.0000x reference)
import jax
import jax.numpy as jnp
from jax.experimental import pallas as pl

B = 128
N = 512
L = 10
LP = 16
TOPK = 20
E = 64
BLP = B * LP
BB = 8
OUTW = L + 2 * E


def _pre_body(x_ref, out_ref):
    out_ref[...] = jnp.zeros((N, BB * LP), jnp.float32)
    for bb in range(BB):
        out_ref[:, bb * LP : bb * LP + L] = x_ref[bb]


def _main_body(xp_ref, s_ref, t_ref, W1_ref, W2_ref, W3_ref, b_ref, agg_ref):
    X = xp_ref[...]
    lane = jax.lax.broadcasted_iota(jnp.int32, (1, BLP), 1) % LP

    def shift(V, s):
        sh = jnp.concatenate([jnp.zeros((N, s), V.dtype), V[:, : BLP - s]], axis=1)
        return jnp.where(lane >= s, sh, 0.0)

    def layer(V, W_ref, bias, d):
        acc = jnp.dot(W_ref[2], V, preferred_element_type=jnp.float32)
        acc += jnp.dot(W_ref[1], shift(V, d), preferred_element_type=jnp.float32)
        acc += jnp.dot(W_ref[0], shift(V, 2 * d), preferred_element_type=jnp.float32)
        return jax.nn.relu(acc + bias)

    b = b_ref[...]
    Z = layer(X, W1_ref, b[0][:, None], 1)
    Z = layer(Z, W2_ref, b[1][:, None], 2)
    Z = layer(Z, W3_ref, b[2][:, None], 4)
    Z = jax.nn.relu(Z + X)

    s = s_ref[...]
    t = t_ref[...]
    ns = s * jax.lax.rsqrt(jnp.sum(s * s, axis=1, keepdims=True))
    nt = t * jax.lax.rsqrt(jnp.sum(t * t, axis=1, keepdims=True))
    c = jax.nn.relu(jnp.dot(nt, ns.T, preferred_element_type=jnp.float32))
    col = jax.lax.broadcasted_iota(jnp.int32, (N, N), 1)
    row = jax.lax.broadcasted_iota(jnp.int32, (N, N), 0)
    S = jnp.where(col == row, -jnp.inf, c)

    A = jnp.zeros((N, N), jnp.float32)
    for _ in range(TOPK):
        v = jnp.max(S, axis=1, keepdims=True)
        m = S == v
        idx = jnp.where(m, col, N)
        jmin = jnp.min(idx, axis=1, keepdims=True)
        first = col == jmin
        A = jnp.where(first, 1.0, A)
        S = jnp.where(first, -jnp.inf, S)

    agg_ref[...] = jax.nn.relu(jnp.dot(A, Z, preferred_element_type=jnp.float32))


def _asm_body(agg_ref, s_ref, t_ref, out_ref):
    se = s_ref[...]
    te = t_ref[...]
    for bb in range(BB):
        out_ref[bb, :, 0:L] = agg_ref[:, bb * LP : bb * LP + L]
        out_ref[bb, :, L : L + E] = se
        out_ref[bb, :, L + E : OUTW] = te


def kernel(x, s_emb, t_emb, W1, b1, W2, b2, W3, b3):
    Ws = [jnp.transpose(W, (2, 0, 1)) for W in (W1, W2, W3)]
    bs = jnp.stack([b1, b2, b3], axis=0)

    xp = pl.pallas_call(
        _pre_body,
        grid=(B // BB,),
        in_specs=[pl.BlockSpec((BB, N, L), lambda i: (i, 0, 0))],
        out_specs=pl.BlockSpec((N, BB * LP), lambda i: (0, i)),
        out_shape=jax.ShapeDtypeStruct((N, BLP), jnp.float32),
    )(x)

    agg = pl.pallas_call(
        _main_body,
        out_shape=jax.ShapeDtypeStruct((N, BLP), jnp.float32),
    )(xp, s_emb, t_emb, Ws[0], Ws[1], Ws[2], bs)

    out3 = pl.pallas_call(
        _asm_body,
        grid=(B // BB,),
        in_specs=[
            pl.BlockSpec((N, BB * LP), lambda i: (0, i)),
            pl.BlockSpec((N, E), lambda i: (0, 0)),
            pl.BlockSpec((N, E), lambda i: (0, 0)),
        ],
        out_specs=pl.BlockSpec((BB, N, OUTW), lambda i: (i, 0, 0)),
        out_shape=jax.ShapeDtypeStruct((B, N, OUTW), jnp.float32),
    )(agg, s_emb, t_emb)

    return out3.reshape(B * N, OUTW)
